# Initial kernel scaffold; baseline (speedup 1.0000x reference)
import jax
import jax.numpy as jnp
from jax.experimental import pallas as pl


def kernel(frag_feature, in_degree, out_degree, W_feat, b_feat, in_tab, out_tab, graph_token):
    raise NotImplementedError("write your pallas kernel here")


# speedup vs baseline: 2.7847x; 2.7847x over previous
import functools

import jax
import jax.numpy as jnp
from jax import lax
from jax.experimental import pallas as pl
from jax.experimental.pallas import tpu as pltpu
from jax.experimental.pallas import tpu_sc as plsc

H = 128
NUM_FRAG = 50
N_CORES = 2
N_SUBCORES = 16
NW = N_CORES * N_SUBCORES
CHUNK = 128


def _sc_embedding_sum(in_tab, out_tab, idx_in, idx_out):
    B = idx_in.shape[0]
    per_w = B // NW
    n_chunks = per_w // CHUNK
    mesh = plsc.VectorSubcoreMesh(core_axis_name="c", subcore_axis_name="s")

    @functools.partial(
        pl.kernel,
        mesh=mesh,
        out_type=jax.ShapeDtypeStruct((B, H), jnp.float32),
        scratch_types=[
            pltpu.VMEM((CHUNK,), jnp.int32),
            pltpu.VMEM((CHUNK,), jnp.int32),
            pltpu.VMEM((CHUNK, H), jnp.float32),
            pltpu.VMEM((CHUNK, H), jnp.float32),
            pltpu.SemaphoreType.DMA,
            pltpu.SemaphoreType.DMA,
        ],
    )
    def k(in_tab_h, out_tab_h, ii_h, io_h, out_h, ii_v, io_v, ri_v, ro_v, s1, s2):
        wid = lax.axis_index("s") * N_CORES + lax.axis_index("c")
        base = wid * per_w

        def chunk_body(ci, carry):
            off = base + ci * CHUNK
            pltpu.sync_copy(ii_h.at[pl.ds(off, CHUNK)], ii_v)
            pltpu.sync_copy(io_h.at[pl.ds(off, CHUNK)], io_v)
            cp1 = pltpu.async_copy(in_tab_h.at[ii_v], ri_v, s1)
            cp2 = pltpu.async_copy(out_tab_h.at[io_v], ro_v, s2)
            cp1.wait()
            cp2.wait()

            def add_row(i, c2):
                for j in range(H // 16):
                    sl = pl.ds(j * 16, 16)
                    ri_v[i, sl] = ri_v[i, sl] + ro_v[i, sl]
                return c2

            lax.fori_loop(0, CHUNK, add_row, 0)
            pltpu.sync_copy(ri_v, out_h.at[pl.ds(off, CHUNK)])
            return carry

        lax.fori_loop(0, n_chunks, chunk_body, 0)

    return k(in_tab, out_tab, idx_in, idx_out)


def _tc_fuse(frag, emb, W, b2, tok, block_g):
    n_graph = frag.shape[0]
    grid = n_graph // block_g

    def body(frag_ref, emb_ref, w_ref, b_ref, tok_ref, out_ref):
        x = frag_ref[...].reshape(block_g * NUM_FRAG, H)
        feat = lax.dot_general(
            x, w_ref[...], (((1,), (1,)), ((), ())),
            preferred_element_type=jnp.float32,
        )
        feat = feat + emb_ref[...].reshape(block_g * NUM_FRAG, H) + b_ref[...]
        tok_rows = jnp.broadcast_to(tok_ref[...][None, :, :], (block_g, 1, H))
        out_ref[...] = jnp.concatenate(
            [tok_rows, feat.reshape(block_g, NUM_FRAG, H)], axis=1)

    return pl.pallas_call(
        body,
        grid=(grid,),
        in_specs=[
            pl.BlockSpec((block_g, NUM_FRAG, H), lambda i: (i, 0, 0)),
            pl.BlockSpec((block_g, NUM_FRAG, H), lambda i: (i, 0, 0)),
            pl.BlockSpec((H, H), lambda i: (0, 0)),
            pl.BlockSpec((1, H), lambda i: (0, 0)),
            pl.BlockSpec((1, H), lambda i: (0, 0)),
        ],
        out_specs=pl.BlockSpec((block_g, NUM_FRAG + 1, H), lambda i: (i, 0, 0)),
        out_shape=jax.ShapeDtypeStruct((n_graph, NUM_FRAG + 1, H), jnp.float32),
    )(frag, emb, W, b2, tok)


def kernel(frag_feature, in_degree, out_degree, W_feat, b_feat, in_tab, out_tab, graph_token):
    n_graph = frag_feature.shape[0]
    idx_in = in_degree.reshape(-1)
    idx_out = out_degree.reshape(-1)
    emb = _sc_embedding_sum(in_tab, out_tab, idx_in, idx_out)
    emb = emb.reshape(n_graph, NUM_FRAG, H)
    return _tc_fuse(frag_feature, emb, W_feat, b_feat.reshape(1, H),
                    graph_token, 64)


# speedup vs baseline: 3.4249x; 1.2299x over previous
import functools

import jax
import jax.numpy as jnp
from jax import lax
from jax.experimental import pallas as pl
from jax.experimental.pallas import tpu as pltpu
from jax.experimental.pallas import tpu_sc as plsc

H = 128
NUM_FRAG = 50
N_CORES = 2
N_SUBCORES = 16
NW = N_CORES * N_SUBCORES
CHUNK = 128
NBUF = 2


def _sc_embedding_sum(in_tab, out_tab, idx_in, idx_out):
    B = idx_in.shape[0]
    per_w = B // NW
    n_chunks = per_w // CHUNK
    mesh = plsc.VectorSubcoreMesh(core_axis_name="c", subcore_axis_name="s")

    @functools.partial(
        pl.kernel,
        mesh=mesh,
        out_type=jax.ShapeDtypeStruct((B, H), jnp.float32),
        scratch_types=[
            pltpu.VMEM_SHARED((512, H), jnp.float32),
            pltpu.VMEM_SHARED((512, H), jnp.float32),
        ]
        + [pltpu.VMEM((CHUNK,), jnp.int32) for _ in range(2 * NBUF)]
        + [pltpu.VMEM((CHUNK, H), jnp.float32) for _ in range(2 * NBUF)]
        + [pltpu.SemaphoreType.DMA for _ in range(2 * NBUF)],
    )
    def k(in_tab_h, out_tab_h, ii_h, io_h, out_h, in_sp, out_sp,
          ii0, ii1, io0, io1, ri0, ri1, ro0, ro1, si0, si1, so0, so1):
        cid = lax.axis_index("c")
        sid = lax.axis_index("s")
        wid = sid * N_CORES + cid
        base = wid * per_w
        ii = [ii0, ii1]
        io = [io0, io1]
        ri = [ri0, ri1]
        ro = [ro0, ro1]
        si = [si0, si1]
        so = [so0, so1]

        @pl.when(sid == 0)
        def _():
            pltpu.sync_copy(in_tab_h, in_sp)
            pltpu.sync_copy(out_tab_h, out_sp)

        plsc.subcore_barrier()

        def fill(b, ci):
            off = base + ci * CHUNK
            pltpu.sync_copy(ii_h.at[pl.ds(off, CHUNK)], ii[b])
            pltpu.sync_copy(io_h.at[pl.ds(off, CHUNK)], io[b])
            pltpu.async_copy(in_sp.at[ii[b]], ri[b], si[b])
            pltpu.async_copy(out_sp.at[io[b]], ro[b], so[b])

        for b in range(NBUF):
            fill(b, b)

        def macro(m, carry):
            for b in range(NBUF):
                ci = NBUF * m + b
                pltpu.make_async_copy(in_sp.at[ii[b]], ri[b], si[b]).wait()
                pltpu.make_async_copy(out_sp.at[io[b]], ro[b], so[b]).wait()

                def add_row(i, c2):
                    for j in range(H // 16):
                        sl = pl.ds(j * 16, 16)
                        ri[b][i, sl] = ri[b][i, sl] + ro[b][i, sl]
                    return c2

                lax.fori_loop(0, CHUNK, add_row, 0)
                off = base + ci * CHUNK
                pltpu.sync_copy(ri[b], out_h.at[pl.ds(off, CHUNK)])

                @pl.when(ci + NBUF < n_chunks)
                def _():
                    fill(b, ci + NBUF)

            return carry

        lax.fori_loop(0, n_chunks // NBUF, macro, 0)

    return k(in_tab, out_tab, idx_in, idx_out)


def _tc_fuse(frag, emb, W, b2, tok, block_g):
    n_graph = frag.shape[0]
    grid = n_graph // block_g

    def body(frag_ref, emb_ref, w_ref, b_ref, tok_ref, out_ref):
        x = frag_ref[...].reshape(block_g * NUM_FRAG, H)
        feat = lax.dot_general(
            x, w_ref[...], (((1,), (1,)), ((), ())),
            preferred_element_type=jnp.float32,
        )
        feat = feat + emb_ref[...].reshape(block_g * NUM_FRAG, H) + b_ref[...]
        tok_rows = jnp.broadcast_to(tok_ref[...][None, :, :], (block_g, 1, H))
        out_ref[...] = jnp.concatenate(
            [tok_rows, feat.reshape(block_g, NUM_FRAG, H)], axis=1)

    return pl.pallas_call(
        body,
        grid=(grid,),
        in_specs=[
            pl.BlockSpec((block_g, NUM_FRAG, H), lambda i: (i, 0, 0)),
            pl.BlockSpec((block_g, NUM_FRAG, H), lambda i: (i, 0, 0)),
            pl.BlockSpec((H, H), lambda i: (0, 0)),
            pl.BlockSpec((1, H), lambda i: (0, 0)),
            pl.BlockSpec((1, H), lambda i: (0, 0)),
        ],
        out_specs=pl.BlockSpec((block_g, NUM_FRAG + 1, H), lambda i: (i, 0, 0)),
        out_shape=jax.ShapeDtypeStruct((n_graph, NUM_FRAG + 1, H), jnp.float32),
    )(frag, emb, W, b2, tok)


def kernel(frag_feature, in_degree, out_degree, W_feat, b_feat, in_tab, out_tab, graph_token):
    n_graph = frag_feature.shape[0]
    idx_in = in_degree.reshape(-1)
    idx_out = out_degree.reshape(-1)
    emb = _sc_embedding_sum(in_tab, out_tab, idx_in, idx_out)
    emb = emb.reshape(n_graph, NUM_FRAG, H)
    return _tc_fuse(frag_feature, emb, W_feat, b_feat.reshape(1, H),
                    graph_token, 64)
